# baseline (speedup 1.0000x reference)
import functools

import jax
import jax.numpy as jnp
from jax import lax
from jax.experimental import pallas as pl
from jax.experimental.pallas import tpu as pltpu
from jax.experimental.pallas import tpu_sc as plsc

S = 2048
H = 2048
FF = 1024
E = 16
TM = 128
NT = 48
NSLOT = NT * TM
EPS = 1e-6
F32 = jnp.float32
NW = 32


def _fiota(shape, dim):
    return jax.lax.broadcasted_iota(jnp.int32, shape, dim).astype(F32)


def _route_kernel(hid_ref, lnw_ref, gwt_ref, xn_ref, st_ref, te_ref,
                  pos1_ref, pos2_ref, wab_ref, c_scr, cum_scr):
    x = hid_ref[:]
    v = jnp.mean(x * x, axis=1, keepdims=True)
    xn = (x * jax.lax.rsqrt(v + EPS)) * lnw_ref[:]
    xn_ref[:] = xn

    logits = jnp.dot(xn.astype(jnp.bfloat16), gwt_ref[:].astype(jnp.bfloat16),
                     preferred_element_type=F32)
    col = _fiota((S, E), 1)

    m1 = jnp.max(logits, axis=1, keepdims=True)
    i1 = jnp.min(jnp.where(logits == m1, col, float(E)), axis=1, keepdims=True)
    o1 = col == i1
    lm = jnp.where(o1, -jnp.inf, logits)
    m2 = jnp.max(lm, axis=1, keepdims=True)
    i2 = jnp.min(jnp.where(lm == m2, col, float(E)), axis=1, keepdims=True)
    o2 = col == i2

    e2 = jnp.exp(m2 - m1)
    wa = 1.0 / (1.0 + e2)
    wb = e2 * wa
    wab_ref[:] = jnp.concatenate([wa, wb], axis=1)

    c_scr[:] = o1.astype(F32) + o2.astype(F32)
    CH = 128
    tri = (_fiota((CH, CH), 0) > _fiota((CH, CH), 1)).astype(F32)

    def chunk_body(c, carry):
        cc = c_scr[pl.ds(c * CH, CH), :]
        within = jnp.dot(tri, cc, preferred_element_type=F32,
                         precision=jax.lax.Precision.HIGHEST)
        cum_scr[pl.ds(c * CH, CH), :] = within + carry
        return carry + jnp.sum(cc, axis=0, keepdims=True)

    counts = jax.lax.fori_loop(0, S // CH, chunk_body,
                               jnp.zeros((1, E), F32))

    padded = jnp.floor((counts + (TM - 1)) / TM) * TM
    tcnt = padded / TM
    er = _fiota((E, E), 0)
    ec = _fiota((E, E), 1)
    pb = jnp.broadcast_to(padded, (E, E))
    off = jnp.sum(jnp.where(ec < er, pb, 0.0), axis=1)
    tb = jnp.broadcast_to(tcnt, (E, E))
    tile_end = jnp.sum(jnp.where(ec <= er, tb, 0.0), axis=1)

    jr = _fiota((NT, E), 0)
    te = jnp.sum((jr >= tile_end[None, :]).astype(F32), axis=1)
    te = jnp.minimum(te, float(E - 1))
    te_ref[:] = te[None, :].astype(jnp.int32)

    cum = cum_scr[:]
    offb = jnp.broadcast_to(off[None, :], (S, E))
    slot1 = (jnp.sum(jnp.where(o1, offb, 0.0), axis=1, keepdims=True)
             + jnp.sum(jnp.where(o1, cum, 0.0), axis=1, keepdims=True))
    slot2 = (jnp.sum(jnp.where(o2, offb, 0.0), axis=1, keepdims=True)
             + jnp.sum(jnp.where(o2, cum, 0.0), axis=1, keepdims=True))
    pos1_ref[:] = slot1.astype(jnp.int32)
    pos2_ref[:] = slot2.astype(jnp.int32)

    tokf = _fiota((S, 1), 0)
    dn = (((0,), (0,)), ((), ()))
    hp = jax.lax.Precision.HIGHEST

    def slot_body(j, _):
        scol = _fiota((S, TM), 1) + (j * TM).astype(F32)
        a1 = (jnp.broadcast_to(slot1, (S, TM)) == scol).astype(F32)
        a2 = (jnp.broadcast_to(slot2, (S, TM)) == scol).astype(F32)
        r = (jax.lax.dot_general(tokf, a1, dn, preferred_element_type=F32,
                                 precision=hp)
             + jax.lax.dot_general(tokf, a2, dn, preferred_element_type=F32,
                                   precision=hp))
        st_ref[pl.ds(j, 1), :] = r.astype(jnp.int32)
        return 0

    jax.lax.fori_loop(0, NT, slot_body, 0)


@functools.lru_cache(maxsize=None)
def _make_row_gather(nrows, gch=32):
    per = nrows // NW
    mesh = plsc.VectorSubcoreMesh(core_axis_name="c", subcore_axis_name="s",
                                  num_cores=2, num_subcores=16)

    @functools.partial(
        pl.kernel, mesh=mesh,
        out_type=jax.ShapeDtypeStruct((nrows, H), F32),
        scratch_types=[
            pltpu.VMEM((gch,), jnp.int32),
            pltpu.VMEM((gch, H), F32),
            pltpu.SemaphoreType.DMA,
        ],
    )
    def gk(table_hbm, idx_hbm, out_hbm, idx_v, rows_v, sem):
        wid = lax.axis_index("s") * 2 + lax.axis_index("c")
        base = wid * per
        for ch in range(per // gch):
            b = base + ch * gch
            pltpu.sync_copy(idx_hbm.at[pl.ds(b, gch)], idx_v)
            pltpu.async_copy(table_hbm.at[idx_v], rows_v, sem).wait()
            pltpu.sync_copy(rows_v, out_hbm.at[pl.ds(b, gch)])

    return gk


def _moe_kernel(te_ref, xs_ref, w13_ref, w2_ref, y_ref):
    gu = jnp.dot(xs_ref[:].astype(jnp.bfloat16), w13_ref[0],
                 preferred_element_type=F32)
    g = gu[:, :FF]
    u = gu[:, FF:]
    h = g * jax.lax.logistic(g) * u
    y_ref[:] = jnp.dot(h.astype(jnp.bfloat16), w2_ref[0],
                       preferred_element_type=F32)


def _shared_kernel(hid_ref, xn_ref, y1_ref, y2_ref, wab_ref,
                   sgt_ref, sut_ref, sdt_ref, out_ref):
    x = xn_ref[:].astype(jnp.bfloat16)
    g = jnp.dot(x, sgt_ref[:], preferred_element_type=F32)
    u = jnp.dot(x, sut_ref[:], preferred_element_type=F32)
    h = g * jax.lax.logistic(g) * u
    sh = jnp.dot(h.astype(jnp.bfloat16), sdt_ref[:],
                 preferred_element_type=F32)
    wa = wab_ref[:, 0:1]
    wb = wab_ref[:, 1:2]
    out_ref[:] = (hid_ref[:] + sh + wa * y1_ref[:] + wb * y2_ref[:])


def kernel(hidden_states, ln_w, gate_w, w13, w2, sg_w, su_w, sd_w):
    Bb, Ss, Hh = hidden_states.shape
    hid = hidden_states.reshape(Ss, Hh)

    xn, slot_tok, tile_exp, pos1, pos2, wab = pl.pallas_call(
        _route_kernel,
        out_shape=[
            jax.ShapeDtypeStruct((S, H), F32),
            jax.ShapeDtypeStruct((NT, TM), jnp.int32),
            jax.ShapeDtypeStruct((1, NT), jnp.int32),
            jax.ShapeDtypeStruct((S, 1), jnp.int32),
            jax.ShapeDtypeStruct((S, 1), jnp.int32),
            jax.ShapeDtypeStruct((S, 2), F32),
        ],
        scratch_shapes=[pltpu.VMEM((S, E), F32), pltpu.VMEM((S, E), F32)],
    )(hid, ln_w.reshape(1, H), gate_w.T)

    xs = _make_row_gather(NSLOT)(xn, slot_tok.reshape(NSLOT))

    grid_spec = pltpu.PrefetchScalarGridSpec(
        num_scalar_prefetch=1,
        grid=(NT,),
        in_specs=[
            pl.BlockSpec((TM, H), lambda j, te: (j, 0)),
            pl.BlockSpec((1, H, 2 * FF), lambda j, te: (te[j], 0, 0)),
            pl.BlockSpec((1, FF, H), lambda j, te: (te[j], 0, 0)),
        ],
        out_specs=pl.BlockSpec((TM, H), lambda j, te: (j, 0)),
    )
    y = pl.pallas_call(
        _moe_kernel,
        grid_spec=grid_spec,
        out_shape=jax.ShapeDtypeStruct((NSLOT, H), F32),
        compiler_params=pltpu.CompilerParams(
            dimension_semantics=("arbitrary",),
        ),
    )(tile_exp.reshape(NT), xs, w13.astype(jnp.bfloat16),
      w2.astype(jnp.bfloat16))

    poscat = jnp.concatenate([pos1.reshape(S), pos2.reshape(S)], axis=0)
    yg = _make_row_gather(2 * S)(y, poscat)

    RB = 256
    NRB = S // RB
    out = pl.pallas_call(
        _shared_kernel,
        grid=(NRB,),
        in_specs=[
            pl.BlockSpec((RB, H), lambda i: (i, 0)),
            pl.BlockSpec((RB, H), lambda i: (i, 0)),
            pl.BlockSpec((RB, H), lambda i: (i, 0)),
            pl.BlockSpec((RB, H), lambda i: (i + NRB, 0)),
            pl.BlockSpec((RB, 2), lambda i: (i, 0)),
            pl.BlockSpec((H, FF), lambda i: (0, 0)),
            pl.BlockSpec((H, FF), lambda i: (0, 0)),
            pl.BlockSpec((FF, H), lambda i: (0, 0)),
        ],
        out_specs=pl.BlockSpec((RB, H), lambda i: (i, 0)),
        out_shape=jax.ShapeDtypeStruct((S, H), F32),
        compiler_params=pltpu.CompilerParams(
            dimension_semantics=("arbitrary",),
        ),
    )(hid, xn, yg, yg, wab, sg_w.T.astype(jnp.bfloat16),
      su_w.T.astype(jnp.bfloat16), sd_w.T.astype(jnp.bfloat16))

    return out.reshape(Bb, Ss, Hh)


# speedup vs baseline: 1.0168x; 1.0168x over previous
import functools

import jax
import jax.numpy as jnp
from jax import lax
from jax.experimental import pallas as pl
from jax.experimental.pallas import tpu as pltpu
from jax.experimental.pallas import tpu_sc as plsc

S = 2048
H = 2048
FF = 1024
E = 16
TM = 128
NT = 48
NSLOT = NT * TM
EPS = 1e-6
F32 = jnp.float32
NW = 32


def _fiota(shape, dim):
    return jax.lax.broadcasted_iota(jnp.int32, shape, dim).astype(F32)


def _route_kernel(hid_ref, lnw_ref, gwt_ref, xn_ref, xnb_ref, st_ref, te_ref,
                  pos1_ref, pos2_ref, wab_ref, c_scr, cum_scr):
    x = hid_ref[:]
    v = jnp.mean(x * x, axis=1, keepdims=True)
    xn = (x * jax.lax.rsqrt(v + EPS)) * lnw_ref[:]
    xn_ref[:] = xn
    xnb_ref[:] = xn.astype(jnp.bfloat16)

    logits = jnp.dot(xn.astype(jnp.bfloat16), gwt_ref[:].astype(jnp.bfloat16),
                     preferred_element_type=F32)
    col = _fiota((S, E), 1)

    m1 = jnp.max(logits, axis=1, keepdims=True)
    i1 = jnp.min(jnp.where(logits == m1, col, float(E)), axis=1, keepdims=True)
    o1 = col == i1
    lm = jnp.where(o1, -jnp.inf, logits)
    m2 = jnp.max(lm, axis=1, keepdims=True)
    i2 = jnp.min(jnp.where(lm == m2, col, float(E)), axis=1, keepdims=True)
    o2 = col == i2

    e2 = jnp.exp(m2 - m1)
    wa = 1.0 / (1.0 + e2)
    wb = e2 * wa
    wab_ref[:] = jnp.concatenate([wa, wb], axis=1)

    c_scr[:] = o1.astype(F32) + o2.astype(F32)
    CH = 128
    tri = (_fiota((CH, CH), 0) > _fiota((CH, CH), 1)).astype(F32)

    def chunk_body(c, carry):
        cc = c_scr[pl.ds(c * CH, CH), :]
        within = jnp.dot(tri, cc, preferred_element_type=F32,
                         precision=jax.lax.Precision.HIGHEST)
        cum_scr[pl.ds(c * CH, CH), :] = within + carry
        return carry + jnp.sum(cc, axis=0, keepdims=True)

    counts = jax.lax.fori_loop(0, S // CH, chunk_body,
                               jnp.zeros((1, E), F32))

    padded = jnp.floor((counts + (TM - 1)) / TM) * TM
    tcnt = padded / TM
    er = _fiota((E, E), 0)
    ec = _fiota((E, E), 1)
    pb = jnp.broadcast_to(padded, (E, E))
    off = jnp.sum(jnp.where(ec < er, pb, 0.0), axis=1)
    tb = jnp.broadcast_to(tcnt, (E, E))
    tile_end = jnp.sum(jnp.where(ec <= er, tb, 0.0), axis=1)

    jr = _fiota((NT, E), 0)
    te = jnp.sum((jr >= tile_end[None, :]).astype(F32), axis=1)
    te = jnp.minimum(te, float(E - 1))
    te_ref[:] = te[None, :].astype(jnp.int32)

    cum = cum_scr[:]
    offb = jnp.broadcast_to(off[None, :], (S, E))
    slot1 = (jnp.sum(jnp.where(o1, offb, 0.0), axis=1, keepdims=True)
             + jnp.sum(jnp.where(o1, cum, 0.0), axis=1, keepdims=True))
    slot2 = (jnp.sum(jnp.where(o2, offb, 0.0), axis=1, keepdims=True)
             + jnp.sum(jnp.where(o2, cum, 0.0), axis=1, keepdims=True))
    pos1_ref[:] = slot1.astype(jnp.int32)
    pos2_ref[:] = slot2.astype(jnp.int32)

    tokf = _fiota((S, 1), 0)
    dn = (((0,), (0,)), ((), ()))
    hp = jax.lax.Precision.HIGHEST

    def slot_body(j, _):
        scol = _fiota((S, TM), 1) + (j * TM).astype(F32)
        a1 = (jnp.broadcast_to(slot1, (S, TM)) == scol).astype(F32)
        a2 = (jnp.broadcast_to(slot2, (S, TM)) == scol).astype(F32)
        r = (jax.lax.dot_general(tokf, a1, dn, preferred_element_type=F32,
                                 precision=hp)
             + jax.lax.dot_general(tokf, a2, dn, preferred_element_type=F32,
                                   precision=hp))
        st_ref[pl.ds(j, 1), :] = r.astype(jnp.int32)
        return 0

    jax.lax.fori_loop(0, NT, slot_body, 0)


@functools.lru_cache(maxsize=None)
def _make_row_gather(nrows, dtype=F32, gch=32):
    per = nrows // NW
    mesh = plsc.VectorSubcoreMesh(core_axis_name="c", subcore_axis_name="s",
                                  num_cores=2, num_subcores=16)

    @functools.partial(
        pl.kernel, mesh=mesh,
        out_type=jax.ShapeDtypeStruct((nrows, H), dtype),
        scratch_types=[
            pltpu.VMEM((gch,), jnp.int32),
            pltpu.VMEM((gch, H), dtype),
            pltpu.SemaphoreType.DMA,
        ],
    )
    def gk(table_hbm, idx_hbm, out_hbm, idx_v, rows_v, sem):
        wid = lax.axis_index("s") * 2 + lax.axis_index("c")
        base = wid * per
        for ch in range(per // gch):
            b = base + ch * gch
            pltpu.sync_copy(idx_hbm.at[pl.ds(b, gch)], idx_v)
            pltpu.async_copy(table_hbm.at[idx_v], rows_v, sem).wait()
            pltpu.sync_copy(rows_v, out_hbm.at[pl.ds(b, gch)])

    return gk


def _moe_kernel(te_ref, xs_ref, w13_ref, w2_ref, y_ref):
    gu = jnp.dot(xs_ref[:].astype(jnp.bfloat16), w13_ref[0],
                 preferred_element_type=F32)
    g = gu[:, :FF]
    u = gu[:, FF:]
    h = g * jax.lax.logistic(g) * u
    y_ref[:] = jnp.dot(h.astype(jnp.bfloat16), w2_ref[0],
                       preferred_element_type=F32)


def _shared_kernel(hid_ref, xnb_ref, sgt_ref, sut_ref, sdt_ref, sh_ref):
    x = xnb_ref[:]
    g = jnp.dot(x, sgt_ref[:], preferred_element_type=F32)
    u = jnp.dot(x, sut_ref[:], preferred_element_type=F32)
    h = g * jax.lax.logistic(g) * u
    sh = jnp.dot(h.astype(jnp.bfloat16), sdt_ref[:],
                 preferred_element_type=F32)
    sh_ref[:] = hid_ref[:] + sh


def _combine_kernel(sh_ref, y1_ref, y2_ref, wab_ref, out_ref):
    wa = wab_ref[:, 0:1]
    wb = wab_ref[:, 1:2]
    out_ref[:] = sh_ref[:] + wa * y1_ref[:] + wb * y2_ref[:]


def kernel(hidden_states, ln_w, gate_w, w13, w2, sg_w, su_w, sd_w):
    Bb, Ss, Hh = hidden_states.shape
    hid = hidden_states.reshape(Ss, Hh)

    xn, xnb, slot_tok, tile_exp, pos1, pos2, wab = pl.pallas_call(
        _route_kernel,
        out_shape=[
            jax.ShapeDtypeStruct((S, H), F32),
            jax.ShapeDtypeStruct((S, H), jnp.bfloat16),
            jax.ShapeDtypeStruct((NT, TM), jnp.int32),
            jax.ShapeDtypeStruct((1, NT), jnp.int32),
            jax.ShapeDtypeStruct((S, 1), jnp.int32),
            jax.ShapeDtypeStruct((S, 1), jnp.int32),
            jax.ShapeDtypeStruct((S, 2), F32),
        ],
        scratch_shapes=[pltpu.VMEM((S, E), F32), pltpu.VMEM((S, E), F32)],
    )(hid, ln_w.reshape(1, H), gate_w.T)

    xs = _make_row_gather(NSLOT)(xn, slot_tok.reshape(NSLOT))

    grid_spec = pltpu.PrefetchScalarGridSpec(
        num_scalar_prefetch=1,
        grid=(NT,),
        in_specs=[
            pl.BlockSpec((TM, H), lambda j, te: (j, 0)),
            pl.BlockSpec((1, H, 2 * FF), lambda j, te: (te[j], 0, 0)),
            pl.BlockSpec((1, FF, H), lambda j, te: (te[j], 0, 0)),
        ],
        out_specs=pl.BlockSpec((TM, H), lambda j, te: (j, 0)),
    )
    y = pl.pallas_call(
        _moe_kernel,
        grid_spec=grid_spec,
        out_shape=jax.ShapeDtypeStruct((NSLOT, H), F32),
        compiler_params=pltpu.CompilerParams(
            dimension_semantics=("arbitrary",),
        ),
    )(tile_exp.reshape(NT), xs, w13.astype(jnp.bfloat16),
      w2.astype(jnp.bfloat16))

    poscat = jnp.concatenate([pos1.reshape(S), pos2.reshape(S)], axis=0)
    yg = _make_row_gather(2 * S)(y, poscat)

    RB = 256
    NRB = S // RB
    sh = pl.pallas_call(
        _shared_kernel,
        grid=(NRB,),
        in_specs=[
            pl.BlockSpec((RB, H), lambda i: (i, 0)),
            pl.BlockSpec((RB, H), lambda i: (i, 0)),
            pl.BlockSpec((H, FF), lambda i: (0, 0)),
            pl.BlockSpec((H, FF), lambda i: (0, 0)),
            pl.BlockSpec((FF, H), lambda i: (0, 0)),
        ],
        out_specs=pl.BlockSpec((RB, H), lambda i: (i, 0)),
        out_shape=jax.ShapeDtypeStruct((S, H), F32),
        compiler_params=pltpu.CompilerParams(
            dimension_semantics=("arbitrary",),
        ),
    )(hid, xnb, sg_w.T.astype(jnp.bfloat16),
      su_w.T.astype(jnp.bfloat16), sd_w.T.astype(jnp.bfloat16))

    out = pl.pallas_call(
        _combine_kernel,
        grid=(NRB,),
        in_specs=[
            pl.BlockSpec((RB, H), lambda i: (i, 0)),
            pl.BlockSpec((RB, H), lambda i: (i, 0)),
            pl.BlockSpec((RB, H), lambda i: (i + NRB, 0)),
            pl.BlockSpec((RB, 2), lambda i: (i, 0)),
        ],
        out_specs=pl.BlockSpec((RB, H), lambda i: (i, 0)),
        out_shape=jax.ShapeDtypeStruct((S, H), F32),
        compiler_params=pltpu.CompilerParams(
            dimension_semantics=("arbitrary",),
        ),
    )(sh, yg, yg, wab)

    return out.reshape(Bb, Ss, Hh)


# speedup vs baseline: 1.2578x; 1.2371x over previous
import functools

import jax
import jax.numpy as jnp
from jax import lax
from jax.experimental import pallas as pl
from jax.experimental.pallas import tpu as pltpu
from jax.experimental.pallas import tpu_sc as plsc

S = 2048
H = 2048
FF = 1024
E = 16
TM = 128
NT = 48
NSLOT = NT * TM
EPS = 1e-6
F32 = jnp.float32
NW = 32


def _fiota(shape, dim):
    return jax.lax.broadcasted_iota(jnp.int32, shape, dim).astype(F32)


def _route_kernel(hid_ref, lnw_ref, gwt_ref, xn_ref, xnb_ref, st_ref, te_ref,
                  pos1_ref, pos2_ref, wab_ref, c_scr, cum_scr):
    x = hid_ref[:]
    v = jnp.mean(x * x, axis=1, keepdims=True)
    xn = (x * jax.lax.rsqrt(v + EPS)) * lnw_ref[:]
    xn_ref[:] = xn
    xnb_ref[:] = xn.astype(jnp.bfloat16)

    logits = jnp.dot(xn.astype(jnp.bfloat16), gwt_ref[:].astype(jnp.bfloat16),
                     preferred_element_type=F32)
    col = _fiota((S, E), 1)

    m1 = jnp.max(logits, axis=1, keepdims=True)
    i1 = jnp.min(jnp.where(logits == m1, col, float(E)), axis=1, keepdims=True)
    o1 = col == i1
    lm = jnp.where(o1, -jnp.inf, logits)
    m2 = jnp.max(lm, axis=1, keepdims=True)
    i2 = jnp.min(jnp.where(lm == m2, col, float(E)), axis=1, keepdims=True)
    o2 = col == i2

    e2 = jnp.exp(m2 - m1)
    wa = 1.0 / (1.0 + e2)
    wb = e2 * wa
    wab_ref[:] = jnp.concatenate([wa, wb], axis=1)

    c_scr[:] = o1.astype(F32) + o2.astype(F32)
    CH = 128
    tri = (_fiota((CH, CH), 0) > _fiota((CH, CH), 1)).astype(F32)

    def chunk_body(c, carry):
        cc = c_scr[pl.ds(c * CH, CH), :]
        within = jnp.dot(tri, cc, preferred_element_type=F32,
                         precision=jax.lax.Precision.HIGHEST)
        cum_scr[pl.ds(c * CH, CH), :] = within + carry
        return carry + jnp.sum(cc, axis=0, keepdims=True)

    counts = jax.lax.fori_loop(0, S // CH, chunk_body,
                               jnp.zeros((1, E), F32))

    padded = jnp.floor((counts + (TM - 1)) / TM) * TM
    tcnt = padded / TM
    er = _fiota((E, E), 0)
    ec = _fiota((E, E), 1)
    pb = jnp.broadcast_to(padded, (E, E))
    off = jnp.sum(jnp.where(ec < er, pb, 0.0), axis=1)
    tb = jnp.broadcast_to(tcnt, (E, E))
    tile_end = jnp.sum(jnp.where(ec <= er, tb, 0.0), axis=1)

    jr = _fiota((NT, E), 0)
    te = jnp.sum((jr >= tile_end[None, :]).astype(F32), axis=1)
    te = jnp.minimum(te, float(E - 1))
    te_ref[:] = te[None, :].astype(jnp.int32)

    cum = cum_scr[:]
    offb = jnp.broadcast_to(off[None, :], (S, E))
    slot1 = (jnp.sum(jnp.where(o1, offb, 0.0), axis=1, keepdims=True)
             + jnp.sum(jnp.where(o1, cum, 0.0), axis=1, keepdims=True))
    slot2 = (jnp.sum(jnp.where(o2, offb, 0.0), axis=1, keepdims=True)
             + jnp.sum(jnp.where(o2, cum, 0.0), axis=1, keepdims=True))
    pos1_ref[:] = slot1.astype(jnp.int32)
    pos2_ref[:] = slot2.astype(jnp.int32)

    tokf = _fiota((S, 1), 0)
    dn = (((0,), (0,)), ((), ()))
    hp = jax.lax.Precision.HIGHEST

    def slot_body(j, _):
        scol = _fiota((S, TM), 1) + (j * TM).astype(F32)
        a1 = (jnp.broadcast_to(slot1, (S, TM)) == scol).astype(F32)
        a2 = (jnp.broadcast_to(slot2, (S, TM)) == scol).astype(F32)
        r = (jax.lax.dot_general(tokf, a1, dn, preferred_element_type=F32,
                                 precision=hp)
             + jax.lax.dot_general(tokf, a2, dn, preferred_element_type=F32,
                                   precision=hp))
        matched = (jnp.sum(a1, axis=0, keepdims=True)
                   + jnp.sum(a2, axis=0, keepdims=True))
        fill = (_fiota((1, TM), 1)
                + ((j % (S // TM)) * TM).astype(F32))
        st_ref[pl.ds(j, 1), :] = (r + (1.0 - matched) * fill).astype(jnp.int32)
        return 0

    jax.lax.fori_loop(0, NT, slot_body, 0)


@functools.lru_cache(maxsize=None)
def _make_row_gather(nrows, dtype=F32, gch=32):
    per = nrows // NW
    mesh = plsc.VectorSubcoreMesh(core_axis_name="c", subcore_axis_name="s",
                                  num_cores=2, num_subcores=16)

    @functools.partial(
        pl.kernel, mesh=mesh,
        out_type=jax.ShapeDtypeStruct((nrows, H), dtype),
        scratch_types=[
            pltpu.VMEM((gch,), jnp.int32),
            pltpu.VMEM((gch, H), dtype),
            pltpu.SemaphoreType.DMA,
        ],
    )
    def gk(table_hbm, idx_hbm, out_hbm, idx_v, rows_v, sem):
        wid = lax.axis_index("s") * 2 + lax.axis_index("c")
        base = wid * per
        for ch in range(per // gch):
            b = base + ch * gch
            pltpu.sync_copy(idx_hbm.at[pl.ds(b, gch)], idx_v)
            pltpu.async_copy(table_hbm.at[idx_v], rows_v, sem).wait()
            pltpu.sync_copy(rows_v, out_hbm.at[pl.ds(b, gch)])

    return gk


def _moe_kernel(te_ref, xs_ref, w13_ref, w2_ref, y_ref):
    gu = jnp.dot(xs_ref[:].astype(jnp.bfloat16), w13_ref[0],
                 preferred_element_type=F32)
    g = gu[:, :FF]
    u = gu[:, FF:]
    h = g * jax.lax.logistic(g) * u
    y_ref[:] = jnp.dot(h.astype(jnp.bfloat16), w2_ref[0],
                       preferred_element_type=F32)


def _shared_kernel(hid_ref, xnb_ref, sgt_ref, sut_ref, sdt_ref, sh_ref):
    x = xnb_ref[:]
    g = jnp.dot(x, sgt_ref[:], preferred_element_type=F32)
    u = jnp.dot(x, sut_ref[:], preferred_element_type=F32)
    h = g * jax.lax.logistic(g) * u
    sh = jnp.dot(h.astype(jnp.bfloat16), sdt_ref[:],
                 preferred_element_type=F32)
    sh_ref[:] = hid_ref[:] + sh


def _combine_kernel(sh_ref, y1_ref, y2_ref, wab_ref, out_ref):
    wa = wab_ref[:, 0:1]
    wb = wab_ref[:, 1:2]
    out_ref[:] = sh_ref[:] + wa * y1_ref[:] + wb * y2_ref[:]


def kernel(hidden_states, ln_w, gate_w, w13, w2, sg_w, su_w, sd_w):
    Bb, Ss, Hh = hidden_states.shape
    hid = hidden_states.reshape(Ss, Hh)

    xn, xnb, slot_tok, tile_exp, pos1, pos2, wab = pl.pallas_call(
        _route_kernel,
        out_shape=[
            jax.ShapeDtypeStruct((S, H), F32),
            jax.ShapeDtypeStruct((S, H), jnp.bfloat16),
            jax.ShapeDtypeStruct((NT, TM), jnp.int32),
            jax.ShapeDtypeStruct((1, NT), jnp.int32),
            jax.ShapeDtypeStruct((S, 1), jnp.int32),
            jax.ShapeDtypeStruct((S, 1), jnp.int32),
            jax.ShapeDtypeStruct((S, 2), F32),
        ],
        scratch_shapes=[pltpu.VMEM((S, E), F32), pltpu.VMEM((S, E), F32)],
    )(hid, ln_w.reshape(1, H), gate_w.T)

    xs = _make_row_gather(NSLOT)(xn, slot_tok.reshape(NSLOT))

    grid_spec = pltpu.PrefetchScalarGridSpec(
        num_scalar_prefetch=1,
        grid=(NT,),
        in_specs=[
            pl.BlockSpec((TM, H), lambda j, te: (j, 0)),
            pl.BlockSpec((1, H, 2 * FF), lambda j, te: (te[j], 0, 0)),
            pl.BlockSpec((1, FF, H), lambda j, te: (te[j], 0, 0)),
        ],
        out_specs=pl.BlockSpec((TM, H), lambda j, te: (j, 0)),
    )
    y = pl.pallas_call(
        _moe_kernel,
        grid_spec=grid_spec,
        out_shape=jax.ShapeDtypeStruct((NSLOT, H), F32),
        compiler_params=pltpu.CompilerParams(
            dimension_semantics=("arbitrary",),
        ),
    )(tile_exp.reshape(NT), xs, w13.astype(jnp.bfloat16),
      w2.astype(jnp.bfloat16))

    poscat = jnp.concatenate([pos1.reshape(S), pos2.reshape(S)], axis=0)
    yg = _make_row_gather(2 * S)(y, poscat)

    RB = 256
    NRB = S // RB
    sh = pl.pallas_call(
        _shared_kernel,
        grid=(NRB,),
        in_specs=[
            pl.BlockSpec((RB, H), lambda i: (i, 0)),
            pl.BlockSpec((RB, H), lambda i: (i, 0)),
            pl.BlockSpec((H, FF), lambda i: (0, 0)),
            pl.BlockSpec((H, FF), lambda i: (0, 0)),
            pl.BlockSpec((FF, H), lambda i: (0, 0)),
        ],
        out_specs=pl.BlockSpec((RB, H), lambda i: (i, 0)),
        out_shape=jax.ShapeDtypeStruct((S, H), F32),
        compiler_params=pltpu.CompilerParams(
            dimension_semantics=("arbitrary",),
        ),
    )(hid, xnb, sg_w.T.astype(jnp.bfloat16),
      su_w.T.astype(jnp.bfloat16), sd_w.T.astype(jnp.bfloat16))

    out = pl.pallas_call(
        _combine_kernel,
        grid=(NRB,),
        in_specs=[
            pl.BlockSpec((RB, H), lambda i: (i, 0)),
            pl.BlockSpec((RB, H), lambda i: (i, 0)),
            pl.BlockSpec((RB, H), lambda i: (i + NRB, 0)),
            pl.BlockSpec((RB, 2), lambda i: (i, 0)),
        ],
        out_specs=pl.BlockSpec((RB, H), lambda i: (i, 0)),
        out_shape=jax.ShapeDtypeStruct((S, H), F32),
        compiler_params=pltpu.CompilerParams(
            dimension_semantics=("arbitrary",),
        ),
    )(sh, yg, yg, wab)

    return out.reshape(Bb, Ss, Hh)


# speedup vs baseline: 1.4073x; 1.1189x over previous
import functools

import jax
import jax.numpy as jnp
from jax import lax
from jax.experimental import pallas as pl
from jax.experimental.pallas import tpu as pltpu
from jax.experimental.pallas import tpu_sc as plsc

S = 2048
H = 2048
FF = 1024
E = 16
TM = 128
NT = 48
NSLOT = NT * TM
EPS = 1e-6
F32 = jnp.float32
NW = 32


def _fiota(shape, dim):
    return jax.lax.broadcasted_iota(jnp.int32, shape, dim).astype(F32)


def _route_kernel(hid_ref, lnw_ref, gwt_ref, xn_ref, xnb_ref, te_ref,
                  pos1_ref, pos2_ref, wab_ref, c_scr, cum_scr):
    x = hid_ref[:]
    v = jnp.mean(x * x, axis=1, keepdims=True)
    xn = (x * jax.lax.rsqrt(v + EPS)) * lnw_ref[:]
    xn_ref[:] = xn
    xnb_ref[:] = xn.astype(jnp.bfloat16)

    logits = jnp.dot(xn.astype(jnp.bfloat16), gwt_ref[:].astype(jnp.bfloat16),
                     preferred_element_type=F32)
    col = _fiota((S, E), 1)

    m1 = jnp.max(logits, axis=1, keepdims=True)
    i1 = jnp.min(jnp.where(logits == m1, col, float(E)), axis=1, keepdims=True)
    o1 = col == i1
    lm = jnp.where(o1, -jnp.inf, logits)
    m2 = jnp.max(lm, axis=1, keepdims=True)
    i2 = jnp.min(jnp.where(lm == m2, col, float(E)), axis=1, keepdims=True)
    o2 = col == i2

    e2 = jnp.exp(m2 - m1)
    wa = 1.0 / (1.0 + e2)
    wb = e2 * wa
    wab_ref[:] = jnp.concatenate([wa, wb], axis=1)

    c_scr[:] = o1.astype(F32) + o2.astype(F32)
    CH = 128
    tri = (_fiota((CH, CH), 0) > _fiota((CH, CH), 1)).astype(F32)

    def chunk_body(c, carry):
        cc = c_scr[pl.ds(c * CH, CH), :]
        within = jnp.dot(tri, cc, preferred_element_type=F32,
                         precision=jax.lax.Precision.HIGHEST)
        cum_scr[pl.ds(c * CH, CH), :] = within + carry
        return carry + jnp.sum(cc, axis=0, keepdims=True)

    counts = jax.lax.fori_loop(0, S // CH, chunk_body,
                               jnp.zeros((1, E), F32))

    padded = jnp.floor((counts + (TM - 1)) / TM) * TM
    tcnt = padded / TM
    er = _fiota((E, E), 0)
    ec = _fiota((E, E), 1)
    pb = jnp.broadcast_to(padded, (E, E))
    off = jnp.sum(jnp.where(ec < er, pb, 0.0), axis=1)
    tb = jnp.broadcast_to(tcnt, (E, E))
    tile_end = jnp.sum(jnp.where(ec <= er, tb, 0.0), axis=1)

    jr = _fiota((NT, E), 0)
    te = jnp.sum((jr >= tile_end[None, :]).astype(F32), axis=1)
    te = jnp.minimum(te, float(E - 1))
    te_ref[:] = te[None, :].astype(jnp.int32)

    cum = cum_scr[:]
    offb = jnp.broadcast_to(off[None, :], (S, E))
    slot1 = (jnp.sum(jnp.where(o1, offb, 0.0), axis=1, keepdims=True)
             + jnp.sum(jnp.where(o1, cum, 0.0), axis=1, keepdims=True))
    slot2 = (jnp.sum(jnp.where(o2, offb, 0.0), axis=1, keepdims=True)
             + jnp.sum(jnp.where(o2, cum, 0.0), axis=1, keepdims=True))
    pos1_ref[:] = slot1.astype(jnp.int32)
    pos2_ref[:] = slot2.astype(jnp.int32)


@functools.lru_cache(maxsize=None)
def _make_row_gather(nrows, dtype=F32, gch=32):
    per = nrows // NW
    mesh = plsc.VectorSubcoreMesh(core_axis_name="c", subcore_axis_name="s",
                                  num_cores=2, num_subcores=16)

    @functools.partial(
        pl.kernel, mesh=mesh,
        out_type=jax.ShapeDtypeStruct((nrows, H), dtype),
        scratch_types=[
            pltpu.VMEM((gch,), jnp.int32),
            pltpu.VMEM((gch, H), dtype),
            pltpu.SemaphoreType.DMA,
        ],
    )
    def gk(table_hbm, idx_hbm, out_hbm, idx_v, rows_v, sem):
        wid = lax.axis_index("s") * 2 + lax.axis_index("c")
        base = wid * per
        for ch in range(per // gch):
            b = base + ch * gch
            pltpu.sync_copy(idx_hbm.at[pl.ds(b, gch)], idx_v)
            pltpu.async_copy(table_hbm.at[idx_v], rows_v, sem).wait()
            pltpu.sync_copy(rows_v, out_hbm.at[pl.ds(b, gch)])

    return gk


@functools.lru_cache(maxsize=None)
def _make_row_scatter(ntok, gch=32):
    per = ntok // NW
    mesh = plsc.VectorSubcoreMesh(core_axis_name="c", subcore_axis_name="s",
                                  num_cores=2, num_subcores=16)

    @functools.partial(
        pl.kernel, mesh=mesh,
        out_type=jax.ShapeDtypeStruct((NSLOT, H), F32),
        scratch_types=[
            pltpu.VMEM((gch,), jnp.int32),
            pltpu.VMEM((gch, H), F32),
            pltpu.SemaphoreType.DMA,
        ],
    )
    def sk(table_hbm, idx_hbm, out_hbm, idx_v, rows_v, sem):
        wid = lax.axis_index("s") * 2 + lax.axis_index("c")
        base = wid * per
        for ch in range(per // gch):
            b = base + ch * gch
            src = lax.rem(b, S)
            pltpu.sync_copy(table_hbm.at[pl.ds(src, gch)], rows_v)
            pltpu.sync_copy(idx_hbm.at[pl.ds(b, gch)], idx_v)
            pltpu.async_copy(rows_v, out_hbm.at[idx_v], sem).wait()

    return sk


def _moe_kernel(te_ref, xs_ref, w13_ref, w2_ref, y_ref):
    gu = jnp.dot(xs_ref[:].astype(jnp.bfloat16), w13_ref[0],
                 preferred_element_type=F32)
    g = gu[:, :FF]
    u = gu[:, FF:]
    h = g * jax.lax.logistic(g) * u
    y_ref[:] = jnp.dot(h.astype(jnp.bfloat16), w2_ref[0],
                       preferred_element_type=F32)


def _shared_kernel(hid_ref, xnb_ref, sgt_ref, sut_ref, sdt_ref, sh_ref):
    x = xnb_ref[:]
    g = jnp.dot(x, sgt_ref[:], preferred_element_type=F32)
    u = jnp.dot(x, sut_ref[:], preferred_element_type=F32)
    h = g * jax.lax.logistic(g) * u
    sh = jnp.dot(h.astype(jnp.bfloat16), sdt_ref[:],
                 preferred_element_type=F32)
    sh_ref[:] = hid_ref[:] + sh


def _combine_kernel(sh_ref, y1_ref, y2_ref, wab_ref, out_ref):
    wa = wab_ref[:, 0:1]
    wb = wab_ref[:, 1:2]
    out_ref[:] = sh_ref[:] + wa * y1_ref[:] + wb * y2_ref[:]


def kernel(hidden_states, ln_w, gate_w, w13, w2, sg_w, su_w, sd_w):
    Bb, Ss, Hh = hidden_states.shape
    hid = hidden_states.reshape(Ss, Hh)

    xn, xnb, tile_exp, pos1, pos2, wab = pl.pallas_call(
        _route_kernel,
        out_shape=[
            jax.ShapeDtypeStruct((S, H), F32),
            jax.ShapeDtypeStruct((S, H), jnp.bfloat16),
            jax.ShapeDtypeStruct((1, NT), jnp.int32),
            jax.ShapeDtypeStruct((S, 1), jnp.int32),
            jax.ShapeDtypeStruct((S, 1), jnp.int32),
            jax.ShapeDtypeStruct((S, 2), F32),
        ],
        scratch_shapes=[pltpu.VMEM((S, E), F32), pltpu.VMEM((S, E), F32)],
    )(hid, ln_w.reshape(1, H), gate_w.T)

    poscat = jnp.concatenate([pos1.reshape(S), pos2.reshape(S)], axis=0)
    xs = _make_row_scatter(2 * S)(xn, poscat)

    grid_spec = pltpu.PrefetchScalarGridSpec(
        num_scalar_prefetch=1,
        grid=(NT,),
        in_specs=[
            pl.BlockSpec((TM, H), lambda j, te: (j, 0)),
            pl.BlockSpec((1, H, 2 * FF), lambda j, te: (te[j], 0, 0)),
            pl.BlockSpec((1, FF, H), lambda j, te: (te[j], 0, 0)),
        ],
        out_specs=pl.BlockSpec((TM, H), lambda j, te: (j, 0)),
    )
    y = pl.pallas_call(
        _moe_kernel,
        grid_spec=grid_spec,
        out_shape=jax.ShapeDtypeStruct((NSLOT, H), F32),
        compiler_params=pltpu.CompilerParams(
            dimension_semantics=("arbitrary",),
        ),
    )(tile_exp.reshape(NT), xs, w13.astype(jnp.bfloat16),
      w2.astype(jnp.bfloat16))

    yg = _make_row_gather(2 * S)(y, poscat)

    RB = 256
    NRB = S // RB
    sh = pl.pallas_call(
        _shared_kernel,
        grid=(NRB,),
        in_specs=[
            pl.BlockSpec((RB, H), lambda i: (i, 0)),
            pl.BlockSpec((RB, H), lambda i: (i, 0)),
            pl.BlockSpec((H, FF), lambda i: (0, 0)),
            pl.BlockSpec((H, FF), lambda i: (0, 0)),
            pl.BlockSpec((FF, H), lambda i: (0, 0)),
        ],
        out_specs=pl.BlockSpec((RB, H), lambda i: (i, 0)),
        out_shape=jax.ShapeDtypeStruct((S, H), F32),
        compiler_params=pltpu.CompilerParams(
            dimension_semantics=("arbitrary",),
        ),
    )(hid, xnb, sg_w.T.astype(jnp.bfloat16),
      su_w.T.astype(jnp.bfloat16), sd_w.T.astype(jnp.bfloat16))

    out = pl.pallas_call(
        _combine_kernel,
        grid=(NRB,),
        in_specs=[
            pl.BlockSpec((RB, H), lambda i: (i, 0)),
            pl.BlockSpec((RB, H), lambda i: (i, 0)),
            pl.BlockSpec((RB, H), lambda i: (i + NRB, 0)),
            pl.BlockSpec((RB, 2), lambda i: (i, 0)),
        ],
        out_specs=pl.BlockSpec((RB, H), lambda i: (i, 0)),
        out_shape=jax.ShapeDtypeStruct((S, H), F32),
        compiler_params=pltpu.CompilerParams(
            dimension_semantics=("arbitrary",),
        ),
    )(sh, yg, yg, wab)

    return out.reshape(Bb, Ss, Hh)
